# Initial kernel scaffold; baseline (speedup 1.0000x reference)
import functools

import jax
import jax.numpy as jnp
from jax import lax
from jax.experimental import pallas as pl
from jax.experimental.pallas import tpu as pltpu
from jax.experimental.pallas import tpu_sc as plsc

_L = 16
_K = 128
_WB = 80


def _sc_agg_body(ncores, nsub, N, D, P, E, npad,
                 h_hbm, edges_hbm, out_hbm,
                 src_idx, dst_idx, rows, ones_v, zrow, zdeg, wb, degv,
                 acc_sh, deg_sh, sem):
    c = lax.axis_index("c")
    s = lax.axis_index("s")
    zero = jnp.zeros((_L,), jnp.float32)
    one = jnp.ones((_L,), jnp.float32)
    rpt = npad // nsub
    nvec = D // _L

    def fill_ones(i, _):
        ones_v[i, :] = one
        return 0
    lax.fori_loop(0, _K, fill_ones, 0)

    def fill_zrow(i, _):
        for j in range(nvec):
            zrow[i, pl.ds(j * _L, _L)] = zero
        return 0
    lax.fori_loop(0, 128, fill_zrow, 0)

    def fill_zdeg(i, _):
        zdeg[i, :] = zero
        return 0
    lax.fori_loop(0, rpt, fill_zdeg, 0)

    C = E // _K
    rem = C % nsub
    nrounds = P // ncores

    for r in range(nrounds):
        m = c * nrounds + r

        for j in range(rpt // 128):
            pltpu.sync_copy(zrow, acc_sh.at[pl.ds(s * rpt + j * 128, 128)])
        pltpu.sync_copy(zdeg, deg_sh.at[pl.ds(s * rpt, rpt)])
        plsc.subcore_barrier()

        nch = (C // nsub) + jnp.where(s < rem, 1, 0)

        def chunk(i, _):
            off = (s + i * nsub) * _K
            pltpu.sync_copy(edges_hbm.at[2 * m, pl.ds(off, _K)], src_idx)
            pltpu.sync_copy(edges_hbm.at[2 * m + 1, pl.ds(off, _K)], dst_idx)
            pltpu.async_copy(h_hbm.at[src_idx], rows, sem).wait()
            pltpu.sync_copy(rows, acc_sh.at[dst_idx], add=True)
            pltpu.sync_copy(ones_v, deg_sh.at[dst_idx], add=True)
            return 0
        lax.fori_loop(0, nch, chunk, 0)
        plsc.subcore_barrier()

        base = s * rpt
        nrows = jnp.minimum(rpt, N - base)
        nwb = (nrows + _WB - 1) // _WB

        def wbody(j, _):
            r0 = base + j * _WB
            pltpu.sync_copy(acc_sh.at[pl.ds(r0, _WB)], wb)
            pltpu.sync_copy(deg_sh.at[pl.ds(r0, _WB)], degv)

            def rbody(i, _):
                rcp = 1.0 / (degv[i, :] + 1e-8)
                for jj in range(nvec):
                    wb[i, pl.ds(jj * _L, _L)] = wb[i, pl.ds(jj * _L, _L)] * rcp
                return 0
            lax.fori_loop(0, _WB, rbody, 0)
            pltpu.sync_copy(wb, out_hbm.at[m, pl.ds(r0, _WB)])
            return 0
        lax.fori_loop(0, nwb, wbody, 0)
        plsc.subcore_barrier()


def _sc_aggregate(h, edges2, P):
    N, D = h.shape
    E = edges2.shape[1]
    info = plsc.get_sparse_core_info()
    nc, ns = info.num_cores, info.num_subcores
    npad = ((N + ns * 128 - 1) // (ns * 128)) * (ns * 128)
    rpt = npad // ns
    body = functools.partial(_sc_agg_body, nc, ns, N, D, P, E, npad)
    mesh = plsc.VectorSubcoreMesh(core_axis_name="c", subcore_axis_name="s")
    f = pl.kernel(
        body,
        out_type=jax.ShapeDtypeStruct((P, N, D), jnp.float32),
        mesh=mesh,
        scratch_types=[
            pltpu.VMEM((_K,), jnp.int32),
            pltpu.VMEM((_K,), jnp.int32),
            pltpu.VMEM((_K, D), jnp.float32),
            pltpu.VMEM((_K, _L), jnp.float32),
            pltpu.VMEM((128, D), jnp.float32),
            pltpu.VMEM((rpt, _L), jnp.float32),
            pltpu.VMEM((_WB, D), jnp.float32),
            pltpu.VMEM((_WB, _L), jnp.float32),
            pltpu.VMEM_SHARED((npad, D), jnp.float32),
            pltpu.VMEM_SHARED((npad, _L), jnp.float32),
            pltpu.SemaphoreType.DMA,
        ],
    )
    return f(h, edges2)


def _t1_body(macc_ref, w_ref, a_ref, fcw_ref, fcb_ref, e_ref, s_ref):
    m = pl.program_id(0)
    x = macc_ref[0]
    w = w_ref[0]
    y = lax.dot_general(x, w, (((1,), (1,)), ((), ())),
                        preferred_element_type=jnp.float32)
    a = a_ref[m]
    e = jnp.where(y >= 0.0, y, a * y)
    e_ref[0] = e
    t = jnp.tanh(lax.dot_general(e, fcw_ref[...], (((1,), (1,)), ((), ())),
                                 preferred_element_type=jnp.float32)
                 + fcb_ref[...])
    s_ref[0, 0] = jnp.sum(t, axis=0)


def _t2_body(n_total, s_ref, att_ref, e_ref, z_ref):
    sp = jnp.sum(s_ref[...], axis=1) * (1.0 / n_total)
    logits = jnp.sum(sp * att_ref[...], axis=1)
    mx = jnp.max(logits)
    ew = jnp.exp(logits - mx)
    beta = ew / jnp.sum(ew)
    z_ref[...] = jnp.sum(beta[:, None, None] * e_ref[...], axis=0)


def kernel(h, edge_indices, W_agg, prelu_a, fc_W, fc_b, att):
    N, D = h.shape
    P = edge_indices.shape[0]
    edges2 = edge_indices.reshape(2 * P, edge_indices.shape[2])

    macc = _sc_aggregate(h, edges2, P)

    nb = 10
    bn = N // nb
    e, S = pl.pallas_call(
        _t1_body,
        grid=(P, nb),
        in_specs=[
            pl.BlockSpec((1, bn, D), lambda m, n: (m, n, 0)),
            pl.BlockSpec((1, D, D), lambda m, n: (m, 0, 0)),
            pl.BlockSpec(memory_space=pltpu.SMEM),
            pl.BlockSpec((D, D), lambda m, n: (0, 0)),
            pl.BlockSpec((1, D), lambda m, n: (0, 0)),
        ],
        out_specs=[
            pl.BlockSpec((1, bn, D), lambda m, n: (m, n, 0)),
            pl.BlockSpec((1, 1, D), lambda m, n: (m, n, 0)),
        ],
        out_shape=[
            jax.ShapeDtypeStruct((P, N, D), jnp.float32),
            jax.ShapeDtypeStruct((P, nb, D), jnp.float32),
        ],
    )(macc, W_agg, prelu_a, fc_W, fc_b.reshape(1, D))

    z = pl.pallas_call(
        functools.partial(_t2_body, N),
        grid=(nb,),
        in_specs=[
            pl.BlockSpec((P, nb, D), lambda n: (0, 0, 0)),
            pl.BlockSpec((1, D), lambda n: (0, 0)),
            pl.BlockSpec((P, bn, D), lambda n: (0, n, 0)),
        ],
        out_specs=pl.BlockSpec((bn, D), lambda n: (n, 0)),
        out_shape=jax.ShapeDtypeStruct((N, D), jnp.float32),
    )(S, att, e)
    return z


# speedup vs baseline: 5.3993x; 5.3993x over previous
import functools

import jax
import jax.numpy as jnp
from jax import lax
from jax.experimental import pallas as pl
from jax.experimental.pallas import tpu as pltpu
from jax.experimental.pallas import tpu_sc as plsc

_L = 16
_K = 128
_WB = 80


def _sc_agg_body(ncores, nsub, N, D, P, E,
                 h_hbm, edges_hbm, out_hbm,
                 src_idx, dst_idx, rows, ones_v, wb, degv,
                 acc_sh, deg_sh, sem):
    c = lax.axis_index("c")
    s = lax.axis_index("s")
    zero = jnp.zeros((_L,), jnp.float32)
    one = jnp.ones((_L,), jnp.float32)
    rpt = ((N + nsub * _WB - 1) // (nsub * _WB)) * _WB
    nvec = D // _L

    def fill_ones(i, _):
        ones_v[i, :] = one
        return 0
    lax.fori_loop(0, _K, fill_ones, 0)

    C = E // _K
    rem = C % nsub
    nrounds = P // ncores

    base = s * rpt
    nrows = jnp.maximum(jnp.minimum(rpt, N - base), 0)
    nwb = (nrows + _WB - 1) // _WB

    for r in range(nrounds):
        m = c * nrounds + r

        def fill_z(i, _):
            for j in range(nvec):
                wb[i, pl.ds(j * _L, _L)] = zero
            degv[i, :] = zero
            return 0
        lax.fori_loop(0, _WB, fill_z, 0)

        def zbody(j, _):
            r0 = base + j * _WB
            pltpu.sync_copy(wb, acc_sh.at[pl.ds(r0, _WB)])
            pltpu.sync_copy(degv, deg_sh.at[pl.ds(r0, _WB)])
            return 0
        lax.fori_loop(0, nwb, zbody, 0)
        plsc.subcore_barrier()

        nch = (C // nsub) + jnp.where(s < rem, 1, 0)

        def chunk(i, _):
            off = (s + i * nsub) * _K
            pltpu.sync_copy(edges_hbm.at[2 * m, pl.ds(off, _K)], src_idx)
            pltpu.sync_copy(edges_hbm.at[2 * m + 1, pl.ds(off, _K)], dst_idx)
            pltpu.async_copy(h_hbm.at[src_idx], rows, sem).wait()
            pltpu.sync_copy(rows, acc_sh.at[dst_idx], add=True)
            pltpu.sync_copy(ones_v, deg_sh.at[dst_idx], add=True)
            return 0
        lax.fori_loop(0, nch, chunk, 0)
        plsc.subcore_barrier()

        def wbody(j, _):
            r0 = base + j * _WB
            pltpu.sync_copy(acc_sh.at[pl.ds(r0, _WB)], wb)
            pltpu.sync_copy(deg_sh.at[pl.ds(r0, _WB)], degv)

            def rbody(i, _):
                rcp = 1.0 / (degv[i, :] + 1e-8)
                for jj in range(nvec):
                    wb[i, pl.ds(jj * _L, _L)] = wb[i, pl.ds(jj * _L, _L)] * rcp
                return 0
            lax.fori_loop(0, _WB, rbody, 0)
            pltpu.sync_copy(wb, out_hbm.at[m, pl.ds(r0, _WB)])
            return 0
        lax.fori_loop(0, nwb, wbody, 0)
        plsc.subcore_barrier()


def _sc_aggregate(h, edges2, P):
    N, D = h.shape
    E = edges2.shape[1]
    info = plsc.get_sparse_core_info()
    nc, ns = info.num_cores, info.num_subcores
    body = functools.partial(_sc_agg_body, nc, ns, N, D, P, E)
    mesh = plsc.VectorSubcoreMesh(core_axis_name="c", subcore_axis_name="s")
    f = pl.kernel(
        body,
        out_type=jax.ShapeDtypeStruct((P, N, D), jnp.float32),
        mesh=mesh,
        scratch_types=[
            pltpu.VMEM((_K,), jnp.int32),
            pltpu.VMEM((_K,), jnp.int32),
            pltpu.VMEM((_K, D), jnp.float32),
            pltpu.VMEM((_K, _L), jnp.float32),
            pltpu.VMEM((_WB, D), jnp.float32),
            pltpu.VMEM((_WB, _L), jnp.float32),
            pltpu.VMEM_SHARED((N, D), jnp.float32),
            pltpu.VMEM_SHARED((N, _L), jnp.float32),
            pltpu.SemaphoreType.DMA,
        ],
        compiler_params=pltpu.CompilerParams(use_tc_tiling_on_sc=False),
    )
    return f(h, edges2)


def _t1_body(macc_ref, w_ref, a_ref, fcw_ref, fcb_ref, e_ref, s_ref):
    m = pl.program_id(0)
    n = pl.program_id(1)
    x = macc_ref[0]
    w = w_ref[0]
    y = lax.dot_general(x, w, (((1,), (1,)), ((), ())),
                        preferred_element_type=jnp.float32)
    a = a_ref[m]
    e = jnp.where(y >= 0.0, y, a * y)
    e_ref[0] = e
    t = jnp.tanh(lax.dot_general(e, fcw_ref[...], (((1,), (1,)), ((), ())),
                                 preferred_element_type=jnp.float32)
                 + fcb_ref[...])

    @pl.when(n == 0)
    def _():
        s_ref[...] = jnp.zeros_like(s_ref)
    s_ref[0, 0] += jnp.sum(t, axis=0)


def _t2_body(n_total, s_ref, att_ref, e_ref, z_ref):
    sp = s_ref[...][:, 0, :] * (1.0 / n_total)
    logits = jnp.sum(sp * att_ref[...], axis=1)
    mx = jnp.max(logits)
    ew = jnp.exp(logits - mx)
    beta = ew / jnp.sum(ew)
    z_ref[...] = jnp.sum(beta[:, None, None] * e_ref[...], axis=0)


def kernel(h, edge_indices, W_agg, prelu_a, fc_W, fc_b, att):
    N, D = h.shape
    P = edge_indices.shape[0]
    edges2 = edge_indices.reshape(2 * P, edge_indices.shape[2])

    macc = _sc_aggregate(h, edges2, P)

    nb = 10
    bn = N // nb
    e, S = pl.pallas_call(
        _t1_body,
        grid=(P, nb),
        in_specs=[
            pl.BlockSpec((1, bn, D), lambda m, n: (m, n, 0)),
            pl.BlockSpec((1, D, D), lambda m, n: (m, 0, 0)),
            pl.BlockSpec(memory_space=pltpu.SMEM),
            pl.BlockSpec((D, D), lambda m, n: (0, 0)),
            pl.BlockSpec((1, D), lambda m, n: (0, 0)),
        ],
        out_specs=[
            pl.BlockSpec((1, bn, D), lambda m, n: (m, n, 0)),
            pl.BlockSpec((1, 1, D), lambda m, n: (m, 0, 0)),
        ],
        out_shape=[
            jax.ShapeDtypeStruct((P, N, D), jnp.float32),
            jax.ShapeDtypeStruct((P, 1, D), jnp.float32),
        ],
    )(macc, W_agg, prelu_a, fc_W, fc_b.reshape(1, D))

    z = pl.pallas_call(
        functools.partial(_t2_body, N),
        grid=(nb,),
        in_specs=[
            pl.BlockSpec((P, 1, D), lambda n: (0, 0, 0)),
            pl.BlockSpec((1, D), lambda n: (0, 0)),
            pl.BlockSpec((P, bn, D), lambda n: (0, n, 0)),
        ],
        out_specs=pl.BlockSpec((bn, D), lambda n: (n, 0)),
        out_shape=jax.ShapeDtypeStruct((N, D), jnp.float32),
    )(S, att, e)
    return z


# speedup vs baseline: 9.4641x; 1.7528x over previous
import functools

import jax
import jax.numpy as jnp
from jax import lax
from jax.experimental import pallas as pl
from jax.experimental.pallas import tpu as pltpu
from jax.experimental.pallas import tpu_sc as plsc

_L = 16
_K = 128
_G = 10
_WB = 16


def _sc_agg_body(ncores, nsub, N, D, P, E,
                 h_hbm, edges_hbm, out_hbm,
                 src_idx, dst_idx, rows0, rows1, ones_v, wb, degv,
                 acc_sh, deg_sh, sem0, sem1):
    c = lax.axis_index("c")
    s = lax.axis_index("s")
    zero = jnp.zeros((_L,), jnp.float32)
    one = jnp.ones((_L,), jnp.float32)
    rpt = ((N + nsub * _WB - 1) // (nsub * _WB)) * _WB
    nvec = D // _L

    def fill_ones(i, _):
        ones_v[i, :] = one
        return 0
    lax.fori_loop(0, _K, fill_ones, 0)

    C = E // _K
    NG = C // _G
    grem = NG % nsub
    nrounds = P // ncores

    base = s * rpt
    nrows = jnp.maximum(jnp.minimum(rpt, N - base), 0)
    nwb = (nrows + _WB - 1) // _WB

    for r in range(nrounds):
        m = c * nrounds + r

        def fill_z(i, _):
            for j in range(nvec):
                wb[i, pl.ds(j * _L, _L)] = zero
            degv[i, :] = zero
            return 0
        lax.fori_loop(0, _WB, fill_z, 0)

        def zbody(j, _):
            r0 = base + j * _WB
            pltpu.sync_copy(wb, acc_sh.at[pl.ds(r0, _WB)])
            pltpu.sync_copy(degv, deg_sh.at[pl.ds(r0, _WB)])
            return 0
        lax.fori_loop(0, nwb, zbody, 0)
        plsc.subcore_barrier()

        ngrp = (NG // nsub) + jnp.where(s < grem, 1, 0)

        def grp(i, _):
            gi = s + i * nsub
            pltpu.sync_copy(edges_hbm.at[2 * m, pl.ds(gi * _G, _G)], src_idx)
            pltpu.sync_copy(edges_hbm.at[2 * m + 1, pl.ds(gi * _G, _G)],
                            dst_idx)
            bufs = (rows0, rows1)
            sems = (sem0, sem1)
            pending = pltpu.async_copy(h_hbm.at[src_idx.at[0]], rows0, sem0)
            for j in range(_G):
                if j + 1 < _G:
                    nxt = pltpu.async_copy(h_hbm.at[src_idx.at[j + 1]],
                                           bufs[(j + 1) % 2],
                                           sems[(j + 1) % 2])
                pending.wait()
                pltpu.sync_copy(bufs[j % 2], acc_sh.at[dst_idx.at[j]],
                                add=True)
                pltpu.sync_copy(ones_v, deg_sh.at[dst_idx.at[j]], add=True)
                if j + 1 < _G:
                    pending = nxt
            return 0
        lax.fori_loop(0, ngrp, grp, 0)
        plsc.subcore_barrier()

        def wbody(j, _):
            r0 = base + j * _WB
            pltpu.sync_copy(acc_sh.at[pl.ds(r0, _WB)], wb)
            pltpu.sync_copy(deg_sh.at[pl.ds(r0, _WB)], degv)

            def rbody(i, _):
                rcp = 1.0 / (degv[i, :] + 1e-8)
                for jj in range(nvec):
                    wb[i, pl.ds(jj * _L, _L)] = wb[i, pl.ds(jj * _L, _L)] * rcp
                return 0
            lax.fori_loop(0, _WB, rbody, 0)
            pltpu.sync_copy(wb, out_hbm.at[m, pl.ds(r0, _WB)])
            return 0
        lax.fori_loop(0, nwb, wbody, 0)
        plsc.subcore_barrier()


def _sc_aggregate(h, edges2, P):
    N, D = h.shape
    E = edges2.shape[1] * edges2.shape[2]
    info = plsc.get_sparse_core_info()
    nc, ns = info.num_cores, info.num_subcores
    body = functools.partial(_sc_agg_body, nc, ns, N, D, P, E)
    mesh = plsc.VectorSubcoreMesh(core_axis_name="c", subcore_axis_name="s")
    f = pl.kernel(
        body,
        out_type=jax.ShapeDtypeStruct((P, N, D), jnp.float32),
        mesh=mesh,
        scratch_types=[
            pltpu.VMEM((_G, _K), jnp.int32),
            pltpu.VMEM((_G, _K), jnp.int32),
            pltpu.VMEM((_K, D), jnp.float32),
            pltpu.VMEM((_K, D), jnp.float32),
            pltpu.VMEM((_K, _L), jnp.float32),
            pltpu.VMEM((_WB, D), jnp.float32),
            pltpu.VMEM((_WB, _L), jnp.float32),
            pltpu.VMEM_SHARED((N, D), jnp.float32),
            pltpu.VMEM_SHARED((N, _L), jnp.float32),
            pltpu.SemaphoreType.DMA,
            pltpu.SemaphoreType.DMA,
        ],
        compiler_params=pltpu.CompilerParams(use_tc_tiling_on_sc=False),
    )
    return f(h, edges2)


def _t1_body(macc_ref, w_ref, a_ref, fcw_ref, fcb_ref, e_ref, s_ref):
    m = pl.program_id(0)
    n = pl.program_id(1)
    x = macc_ref[0]
    w = w_ref[0]
    y = lax.dot_general(x, w, (((1,), (1,)), ((), ())),
                        preferred_element_type=jnp.float32)
    a = a_ref[m]
    e = jnp.where(y >= 0.0, y, a * y)
    e_ref[0] = e
    t = jnp.tanh(lax.dot_general(e, fcw_ref[...], (((1,), (1,)), ((), ())),
                                 preferred_element_type=jnp.float32)
                 + fcb_ref[...])

    @pl.when(n == 0)
    def _():
        s_ref[...] = jnp.zeros_like(s_ref)
    s_ref[0, 0] += jnp.sum(t, axis=0)


def _t2_body(n_total, s_ref, att_ref, e_ref, z_ref):
    sp = s_ref[...][:, 0, :] * (1.0 / n_total)
    logits = jnp.sum(sp * att_ref[...], axis=1)
    mx = jnp.max(logits)
    ew = jnp.exp(logits - mx)
    beta = ew / jnp.sum(ew)
    z_ref[...] = jnp.sum(beta[:, None, None] * e_ref[...], axis=0)


def kernel(h, edge_indices, W_agg, prelu_a, fc_W, fc_b, att):
    N, D = h.shape
    P = edge_indices.shape[0]
    E = edge_indices.shape[2]
    edges2 = edge_indices.reshape(2 * P, E // _K, _K)

    macc = _sc_aggregate(h, edges2, P)

    nb = 10
    bn = N // nb
    e, S = pl.pallas_call(
        _t1_body,
        grid=(P, nb),
        in_specs=[
            pl.BlockSpec((1, bn, D), lambda m, n: (m, n, 0)),
            pl.BlockSpec((1, D, D), lambda m, n: (m, 0, 0)),
            pl.BlockSpec(memory_space=pltpu.SMEM),
            pl.BlockSpec((D, D), lambda m, n: (0, 0)),
            pl.BlockSpec((1, D), lambda m, n: (0, 0)),
        ],
        out_specs=[
            pl.BlockSpec((1, bn, D), lambda m, n: (m, n, 0)),
            pl.BlockSpec((1, 1, D), lambda m, n: (m, 0, 0)),
        ],
        out_shape=[
            jax.ShapeDtypeStruct((P, N, D), jnp.float32),
            jax.ShapeDtypeStruct((P, 1, D), jnp.float32),
        ],
    )(macc, W_agg, prelu_a, fc_W, fc_b.reshape(1, D))

    z = pl.pallas_call(
        functools.partial(_t2_body, N),
        grid=(nb,),
        in_specs=[
            pl.BlockSpec((P, 1, D), lambda n: (0, 0, 0)),
            pl.BlockSpec((1, D), lambda n: (0, 0)),
            pl.BlockSpec((P, bn, D), lambda n: (0, n, 0)),
        ],
        out_specs=pl.BlockSpec((bn, D), lambda n: (n, 0)),
        out_shape=jax.ShapeDtypeStruct((N, D), jnp.float32),
    )(S, att, e)
    return z


# speedup vs baseline: 10.6564x; 1.1260x over previous
import functools

import jax
import jax.numpy as jnp
from jax import lax
from jax.experimental import pallas as pl
from jax.experimental.pallas import tpu as pltpu
from jax.experimental.pallas import tpu_sc as plsc

_L = 16
_K = 128
_G = 20
_WB = 80


def _sc_agg_body(ncores, nsub, N, D, P, E,
                 h_hbm, edges_hbm, out_hbm,
                 src_idx, dst_idx, rows0, rows1, ones_v,
                 acc_sh, deg_sh, semg0, semg1, semsc0, semsc1):
    c = lax.axis_index("c")
    s = lax.axis_index("s")
    zero = jnp.zeros((_L,), jnp.float32)
    one = jnp.ones((_L,), jnp.float32)
    rpt = ((N + nsub * _WB - 1) // (nsub * _WB)) * _WB
    nvec = D // _L

    C = E // _K
    NG = C // _G
    grem = NG % nsub
    nrounds = P // ncores

    base = s * rpt
    nrows = jnp.maximum(jnp.minimum(rpt, N - base), 0)
    nwb = (nrows + _WB - 1) // _WB

    wb = rows0.at[pl.ds(0, _WB)]
    degv = ones_v.at[pl.ds(0, _WB)]

    gbufs = (rows0, rows1)
    gsems = (semg0, semg1)
    scsems = (semsc0, semsc1)

    for r in range(nrounds):
        m = c * nrounds + r

        def fill_z(i, _):
            for j in range(nvec):
                rows0[i, pl.ds(j * _L, _L)] = zero
            ones_v[i, :] = zero
            return 0
        lax.fori_loop(0, _WB, fill_z, 0)

        def zbody(j, _):
            r0 = base + j * _WB
            pltpu.sync_copy(wb, acc_sh.at[pl.ds(r0, _WB)])
            pltpu.sync_copy(degv, deg_sh.at[pl.ds(r0, _WB)])
            return 0
        lax.fori_loop(0, nwb, zbody, 0)

        def fill_ones(i, _):
            ones_v[i, :] = one
            return 0
        lax.fori_loop(0, _K, fill_ones, 0)
        plsc.subcore_barrier()

        ngrp = (NG // nsub) + jnp.where(s < grem, 1, 0)

        def grp(i, _):
            gi = s + i * nsub
            pltpu.sync_copy(edges_hbm.at[2 * m, pl.ds(gi * _G, _G)], src_idx)
            pltpu.sync_copy(edges_hbm.at[2 * m + 1, pl.ds(gi * _G, _G)],
                            dst_idx)
            sc = [None] * _G
            pending = pltpu.async_copy(h_hbm.at[src_idx.at[0]], rows0, semg0)
            for j in range(_G):
                if j + 1 < _G:
                    if j >= 1:
                        sc[j - 1].wait()
                    nxt = pltpu.async_copy(h_hbm.at[src_idx.at[j + 1]],
                                           gbufs[(j + 1) % 2],
                                           gsems[(j + 1) % 2])
                pending.wait()
                sc[j] = pltpu.async_copy(gbufs[j % 2],
                                         acc_sh.at[dst_idx.at[j]],
                                         scsems[j % 2], add=True)
                pltpu.sync_copy(ones_v, deg_sh.at[dst_idx.at[j]], add=True)
                if j + 1 < _G:
                    pending = nxt
            if _G >= 2:
                sc[_G - 2].wait()
            sc[_G - 1].wait()
            return 0
        lax.fori_loop(0, ngrp, grp, 0)
        plsc.subcore_barrier()

        def wbody(j, _):
            r0 = base + j * _WB
            pltpu.sync_copy(acc_sh.at[pl.ds(r0, _WB)], wb)
            pltpu.sync_copy(deg_sh.at[pl.ds(r0, _WB)], degv)

            def rbody(i, _):
                rcp = 1.0 / (degv[i, :] + 1e-8)
                for jj in range(nvec):
                    rows0[i, pl.ds(jj * _L, _L)] = (
                        rows0[i, pl.ds(jj * _L, _L)] * rcp)
                return 0
            lax.fori_loop(0, _WB, rbody, 0)
            pltpu.sync_copy(wb, out_hbm.at[m, pl.ds(r0, _WB)])
            return 0
        lax.fori_loop(0, nwb, wbody, 0)
        plsc.subcore_barrier()


def _sc_aggregate(h, edges2, P):
    N, D = h.shape
    E = edges2.shape[1] * edges2.shape[2]
    info = plsc.get_sparse_core_info()
    nc, ns = info.num_cores, info.num_subcores
    body = functools.partial(_sc_agg_body, nc, ns, N, D, P, E)
    mesh = plsc.VectorSubcoreMesh(core_axis_name="c", subcore_axis_name="s")
    f = pl.kernel(
        body,
        out_type=jax.ShapeDtypeStruct((P, N, D), jnp.float32),
        mesh=mesh,
        scratch_types=[
            pltpu.VMEM((_G, _K), jnp.int32),
            pltpu.VMEM((_G, _K), jnp.int32),
            pltpu.VMEM((_K, D), jnp.float32),
            pltpu.VMEM((_K, D), jnp.float32),
            pltpu.VMEM((_K, _L), jnp.float32),
            pltpu.VMEM_SHARED((N, D), jnp.float32),
            pltpu.VMEM_SHARED((N, _L), jnp.float32),
            pltpu.SemaphoreType.DMA,
            pltpu.SemaphoreType.DMA,
            pltpu.SemaphoreType.DMA,
            pltpu.SemaphoreType.DMA,
        ],
        compiler_params=pltpu.CompilerParams(use_tc_tiling_on_sc=False),
    )
    return f(h, edges2)


def _t1_body(macc_ref, w_ref, a_ref, fcw_ref, fcb_ref, e_ref, s_ref):
    m = pl.program_id(0)
    n = pl.program_id(1)
    x = macc_ref[0]
    w = w_ref[0]
    y = lax.dot_general(x, w, (((1,), (1,)), ((), ())),
                        preferred_element_type=jnp.float32)
    a = a_ref[m]
    e = jnp.where(y >= 0.0, y, a * y)
    e_ref[0] = e
    t = jnp.tanh(lax.dot_general(e, fcw_ref[...], (((1,), (1,)), ((), ())),
                                 preferred_element_type=jnp.float32)
                 + fcb_ref[...])

    @pl.when(n == 0)
    def _():
        s_ref[...] = jnp.zeros_like(s_ref)
    s_ref[0, 0] += jnp.sum(t, axis=0)


def _t2_body(n_total, s_ref, att_ref, e_ref, z_ref):
    sp = s_ref[...][:, 0, :] * (1.0 / n_total)
    logits = jnp.sum(sp * att_ref[...], axis=1)
    mx = jnp.max(logits)
    ew = jnp.exp(logits - mx)
    beta = ew / jnp.sum(ew)
    z_ref[...] = jnp.sum(beta[:, None, None] * e_ref[...], axis=0)


def kernel(h, edge_indices, W_agg, prelu_a, fc_W, fc_b, att):
    N, D = h.shape
    P = edge_indices.shape[0]
    E = edge_indices.shape[2]
    edges2 = edge_indices.reshape(2 * P, E // _K, _K)

    macc = _sc_aggregate(h, edges2, P)

    nb = 10
    bn = N // nb
    e, S = pl.pallas_call(
        _t1_body,
        grid=(P, nb),
        in_specs=[
            pl.BlockSpec((1, bn, D), lambda m, n: (m, n, 0)),
            pl.BlockSpec((1, D, D), lambda m, n: (m, 0, 0)),
            pl.BlockSpec(memory_space=pltpu.SMEM),
            pl.BlockSpec((D, D), lambda m, n: (0, 0)),
            pl.BlockSpec((1, D), lambda m, n: (0, 0)),
        ],
        out_specs=[
            pl.BlockSpec((1, bn, D), lambda m, n: (m, n, 0)),
            pl.BlockSpec((1, 1, D), lambda m, n: (m, 0, 0)),
        ],
        out_shape=[
            jax.ShapeDtypeStruct((P, N, D), jnp.float32),
            jax.ShapeDtypeStruct((P, 1, D), jnp.float32),
        ],
    )(macc, W_agg, prelu_a, fc_W, fc_b.reshape(1, D))

    z = pl.pallas_call(
        functools.partial(_t2_body, N),
        grid=(nb,),
        in_specs=[
            pl.BlockSpec((P, 1, D), lambda n: (0, 0, 0)),
            pl.BlockSpec((1, D), lambda n: (0, 0)),
            pl.BlockSpec((P, bn, D), lambda n: (0, n, 0)),
        ],
        out_specs=pl.BlockSpec((bn, D), lambda n: (n, 0)),
        out_shape=jax.ShapeDtypeStruct((N, D), jnp.float32),
    )(S, att, e)
    return z
